# Initial kernel scaffold; baseline (speedup 1.0000x reference)
import jax
import jax.numpy as jnp
from jax.experimental import pallas as pl


def kernel(x_nt1, x_nt2, edge_label_index):
    raise NotImplementedError("write your pallas kernel here")


# speedup vs baseline: 3.5011x; 3.5011x over previous
import functools

import jax
import jax.numpy as jnp
from jax import lax
from jax.experimental import pallas as pl
from jax.experimental.pallas import tpu as pltpu
from jax.experimental.pallas import tpu_sc as plsc

D = 128
E = 320000
NC, NS, L = 2, 16, 16
NW = NC * NS
PER_W = E // NW
C = 80
NCHUNK = PER_W // C


def _sc_kernel(x1_hbm, x2_hbm, i1_hbm, i2_hbm, out_hbm,
               idx1_v, idx2_v, rows1_v, rows2_v, psum_v, outc_v, sem1, sem2):
  wid = lax.axis_index("s") * NC + lax.axis_index("c")
  lane16 = lax.iota(jnp.int32, L) * L

  def chunk_body(c, carry):
    base = wid * PER_W + c * C
    pltpu.sync_copy(i1_hbm.at[pl.ds(base, C)], idx1_v)
    pltpu.sync_copy(i2_hbm.at[pl.ds(base, C)], idx2_v)
    cp1 = pltpu.async_copy(x1_hbm.at[idx1_v], rows1_v, sem1)
    cp2 = pltpu.async_copy(x2_hbm.at[idx2_v], rows2_v, sem2)
    cp1.wait()
    cp2.wait()

    def group_body(g, carry2):
      for j in range(L):
        e = g * L + j
        acc = rows1_v[e, pl.ds(0, L)] * rows2_v[e, pl.ds(0, L)]
        for k in range(1, D // L):
          acc = acc + rows1_v[e, pl.ds(k * L, L)] * rows2_v[e, pl.ds(k * L, L)]
        psum_v[pl.ds(j * L, L)] = acc
      tot = plsc.load_gather(psum_v, [lane16])
      for l in range(1, L):
        tot = tot + plsc.load_gather(psum_v, [lane16 + l])
      outc_v[pl.ds(g * L, L)] = tot
      return carry2

    lax.fori_loop(0, C // L, group_body, 0)
    pltpu.sync_copy(outc_v, out_hbm.at[pl.ds(base, C)])
    return carry

  lax.fori_loop(0, NCHUNK, chunk_body, 0)


@functools.partial(
    pl.kernel,
    mesh=plsc.VectorSubcoreMesh(core_axis_name="c", subcore_axis_name="s"),
    out_type=jax.ShapeDtypeStruct((E,), jnp.float32),
    compiler_params=pltpu.CompilerParams(needs_layout_passes=False),
    scratch_types=[
        pltpu.VMEM((C,), jnp.int32),
        pltpu.VMEM((C,), jnp.int32),
        pltpu.VMEM((C, D), jnp.float32),
        pltpu.VMEM((C, D), jnp.float32),
        pltpu.VMEM((L * L,), jnp.float32),
        pltpu.VMEM((C,), jnp.float32),
        pltpu.SemaphoreType.DMA,
        pltpu.SemaphoreType.DMA,
    ],
)
def _edge_dot(x1, x2, i1, i2, out, *scratch):
  _sc_kernel(x1, x2, i1, i2, out, *scratch)


def kernel(x_nt1, x_nt2, edge_label_index):
  i1 = edge_label_index[0].astype(jnp.int32)
  i2 = edge_label_index[1].astype(jnp.int32)
  return _edge_dot(x_nt1, x_nt2, i1, i2)


# speedup vs baseline: 8.0009x; 2.2852x over previous
import functools

import jax
import jax.numpy as jnp
from jax import lax
from jax.experimental import pallas as pl
from jax.experimental.pallas import tpu as pltpu
from jax.experimental.pallas import tpu_sc as plsc

D = 128
E = 320000
NC, NS, L = 2, 16, 16
NW = NC * NS
PER_W = E // NW
C = 80
NCHUNK = PER_W // C
NPAIR = (NCHUNK - 1) // 2


def _sc_kernel(x1_hbm, x2_hbm, i1_hbm, i2_hbm, out_hbm,
               idx1_v, idx2_v, rA1, rA2, rB1, rB2, psum_v, outw_v,
               si1, si2, sA1, sA2, sB1, sB2):
  wid = lax.axis_index("s") * NC + lax.axis_index("c")
  wbase = wid * PER_W
  lane16 = lax.iota(jnp.int32, L) * L

  cpi1 = pltpu.async_copy(i1_hbm.at[pl.ds(wbase, PER_W)], idx1_v, si1)
  cpi2 = pltpu.async_copy(i2_hbm.at[pl.ds(wbase, PER_W)], idx2_v, si2)
  cpi1.wait()
  cpi2.wait()

  def start(c, r1, r2, s1, s2):
    pltpu.async_copy(x1_hbm.at[idx1_v.at[pl.ds(c * C, C)]], r1, s1)
    pltpu.async_copy(x2_hbm.at[idx2_v.at[pl.ds(c * C, C)]], r2, s2)

  def wait(c, r1, r2, s1, s2):
    pltpu.make_async_copy(x1_hbm.at[idx1_v.at[pl.ds(c * C, C)]], r1, s1).wait()
    pltpu.make_async_copy(x2_hbm.at[idx2_v.at[pl.ds(c * C, C)]], r2, s2).wait()

  def compute(c, r1, r2):
    def group_body(g, carry):
      for j in range(L):
        acc = r1[g * L + j, pl.ds(0, L)] * r2[g * L + j, pl.ds(0, L)]
        for k in range(1, D // L):
          acc = acc + (r1[g * L + j, pl.ds(k * L, L)]
                       * r2[g * L + j, pl.ds(k * L, L)])
        psum_v[pl.ds(j * L, L)] = acc
      tot = plsc.load_gather(psum_v, [lane16])
      for l in range(1, L):
        tot = tot + plsc.load_gather(psum_v, [lane16 + l])
      outw_v[pl.ds(c * C + g * L, L)] = tot
      return carry
    lax.fori_loop(0, C // L, group_body, 0)

  start(0, rA1, rA2, sA1, sA2)

  def pair_body(i, carry):
    c0 = 2 * i
    start(c0 + 1, rB1, rB2, sB1, sB2)
    wait(c0, rA1, rA2, sA1, sA2)
    compute(c0, rA1, rA2)
    start(c0 + 2, rA1, rA2, sA1, sA2)
    wait(c0 + 1, rB1, rB2, sB1, sB2)
    compute(c0 + 1, rB1, rB2)
    return carry

  lax.fori_loop(0, NPAIR, pair_body, 0)
  wait(NCHUNK - 1, rA1, rA2, sA1, sA2)
  compute(NCHUNK - 1, rA1, rA2)

  pltpu.sync_copy(outw_v, out_hbm.at[pl.ds(wbase, PER_W)])


@functools.partial(
    pl.kernel,
    mesh=plsc.VectorSubcoreMesh(core_axis_name="c", subcore_axis_name="s"),
    out_type=jax.ShapeDtypeStruct((E,), jnp.float32),
    compiler_params=pltpu.CompilerParams(needs_layout_passes=False),
    scratch_types=[
        pltpu.VMEM((PER_W,), jnp.int32),
        pltpu.VMEM((PER_W,), jnp.int32),
        pltpu.VMEM((C, D), jnp.float32),
        pltpu.VMEM((C, D), jnp.float32),
        pltpu.VMEM((C, D), jnp.float32),
        pltpu.VMEM((C, D), jnp.float32),
        pltpu.VMEM((L * L,), jnp.float32),
        pltpu.VMEM((PER_W,), jnp.float32),
        pltpu.SemaphoreType.DMA,
        pltpu.SemaphoreType.DMA,
        pltpu.SemaphoreType.DMA,
        pltpu.SemaphoreType.DMA,
        pltpu.SemaphoreType.DMA,
        pltpu.SemaphoreType.DMA,
    ],
)
def _edge_dot(x1, x2, i1, i2, out, *scratch):
  _sc_kernel(x1, x2, i1, i2, out, *scratch)


def kernel(x_nt1, x_nt2, edge_label_index):
  i1 = edge_label_index[0].astype(jnp.int32)
  i2 = edge_label_index[1].astype(jnp.int32)
  return _edge_dot(x_nt1, x_nt2, i1, i2)
